# baseline (speedup 1.0000x reference)
import functools

import jax
import jax.numpy as jnp
from jax import lax
from jax.experimental import pallas as pl
from jax.experimental.pallas import tpu as pltpu
from jax.experimental.pallas import tpu_sc as plsc

N = 10000
E = 320000
D = 128
H = 256

NC = 2
NS = 16
NPAD = 10240
RPW = NPAD // NS
K = 80
EPW_PROP = E // NS
EPW_HALF = E // (NC * NS)

_f32 = jnp.float32


def _mesh():
    return plsc.VectorSubcoreMesh(core_axis_name="c", subcore_axis_name="s")


def _sc_degree(dst):
    @functools.partial(
        pl.kernel,
        mesh=_mesh(),
        out_type=jax.ShapeDtypeStruct((NC * NPAD,), _f32),
        scratch_types=[
            pltpu.VMEM((K,), jnp.int32),
            pltpu.VMEM((K,), _f32),
            pltpu.VMEM((RPW,), _f32),
            pltpu.VMEM_SHARED((NPAD,), _f32),
        ],
    )
    def k(dst_hbm, deg_hbm, dst_v, ones_v, zero_v, acc):
        c = lax.axis_index("c")
        s = lax.axis_index("s")
        wid = s * NC + c

        def fill(i, _):
            ones_v[pl.ds(i * 16, 16)] = jnp.full((16,), 1.0, _f32)
            return 0
        lax.fori_loop(0, K // 16, fill, 0, unroll=False)

        def zf(i, _):
            zero_v[pl.ds(i * 16, 16)] = jnp.zeros((16,), _f32)
            return 0
        lax.fori_loop(0, RPW // 16, zf, 0, unroll=False)
        pltpu.sync_copy(zero_v, acc.at[pl.ds(s * RPW, RPW)])
        plsc.subcore_barrier()

        def chunk(kk, _):
            base = wid * EPW_HALF + kk * K
            pltpu.sync_copy(dst_hbm.at[pl.ds(base, K)], dst_v)
            pltpu.sync_copy(ones_v, acc.at[dst_v], add=True)
            return 0
        lax.fori_loop(0, EPW_HALF // K, chunk, 0, unroll=False)
        plsc.subcore_barrier()
        pltpu.sync_copy(acc.at[pl.ds(s * RPW, RPW)], zero_v)
        pltpu.sync_copy(zero_v, deg_hbm.at[pl.ds(c * NPAD + s * RPW, RPW)])

    return k(dst)


HH = H // 2


IB = 10
KP = 40
NBUF = 5
NBUF_PAIR = 5
TPROP = EPW_PROP // KP


def _sc_propagate(y_lo, y_hi, src2, dst2):
    @functools.partial(
        pl.kernel,
        mesh=_mesh(),
        out_type=jax.ShapeDtypeStruct((NC, NPAD, HH), _f32),
        scratch_types=(
            [pltpu.VMEM((2, IB, KP), jnp.int32),
             pltpu.VMEM((2, IB, KP), jnp.int32)]
            + [pltpu.VMEM((KP, HH), _f32)] * NBUF
            + [pltpu.VMEM_SHARED((NPAD, HH), _f32)]
            + [pltpu.SemaphoreType.DMA] * (2 * NBUF)
        ),
    )
    def k(ylo_hbm, yhi_hbm, src2_hbm, dst2_hbm, z_hbm, srcb, dstb, *rest):
        rows = rest[:NBUF]
        acc = rest[NBUF]
        sg = rest[NBUF + 1:NBUF + 1 + NBUF]
        ss = rest[NBUF + 1 + NBUF:]
        rows0 = rows[0]
        c = lax.axis_index("c")
        s = lax.axis_index("s")

        def zf(i, _):
            rows0[i // 8, pl.ds((i % 8) * 16, 16)] = jnp.zeros((16,), _f32)
            return 0
        lax.fori_loop(0, KP * HH // 16, zf, 0, unroll=8)

        def zcopy(i, _):
            pltpu.sync_copy(rows0, acc.at[pl.ds(s * RPW + i * KP, KP)])
            return 0
        lax.fori_loop(0, RPW // KP, zcopy, 0, unroll=False)
        plsc.subcore_barrier()

        gb0 = s * (TPROP // IB)

        def edge_loop(y_hbm):
            def wait_gather(p, slot, irow):
                pltpu.make_async_copy(
                    y_hbm.at[srcb.at[slot, irow]], rows[p], sg[p]).wait()

            def wait_scatter(p, slot, irow):
                pltpu.make_async_copy(
                    rows[p], acc.at[dstb.at[slot, irow]], ss[p]).wait()

            def group(g, _):
                q = jnp.remainder(g, 2)
                pltpu.sync_copy(src2_hbm.at[gb0 + g], srcb.at[q])
                pltpu.sync_copy(dst2_hbm.at[gb0 + g], dstb.at[q])
                for b in range(IB):
                    kk = g * IB + b
                    p = b % NBUF
                    pm1 = (b - 1) % NBUF
                    pslot, prow = (q, b - 1) if b >= 1 else (1 - q, IB - 1)

                    @pl.when(kk >= NBUF)
                    def _():
                        wait_scatter(p, pslot, prow)
                    pltpu.async_copy(y_hbm.at[srcb.at[q, b]], rows[p], sg[p])

                    @pl.when(kk >= 1)
                    def _():
                        wait_gather(pm1, pslot, prow)
                        pltpu.async_copy(rows[pm1],
                                         acc.at[dstb.at[pslot, prow]],
                                         ss[pm1], add=True)
                return 0
            lax.fori_loop(0, TPROP // IB, group, 0, unroll=False)
            qlast = (TPROP // IB - 1) % 2
            plast = (TPROP - 1) % NBUF
            wait_gather(plast, qlast, IB - 1)
            pltpu.async_copy(rows[plast], acc.at[dstb.at[qlast, IB - 1]],
                             ss[plast], add=True)
            for pp in range(NBUF):
                wait_scatter(pp, qlast, IB - 1)

        @pl.when(c == 0)
        def _():
            edge_loop(ylo_hbm)

        @pl.when(c == 1)
        def _():
            edge_loop(yhi_hbm)

        plsc.subcore_barrier()

        def wout(i, _):
            r0 = s * RPW + i * KP
            pltpu.sync_copy(acc.at[pl.ds(r0, KP)], rows0)
            pltpu.sync_copy(rows0, z_hbm.at[c, pl.ds(r0, KP)])
            return 0
        lax.fori_loop(0, RPW // KP, wout, 0, unroll=False)

    return k(y_lo, y_hi, src2, dst2)


K2 = 40
TPAIR = EPW_HALF // K2


def _sc_pair(a, b, src4, dst4):
    @functools.partial(
        pl.kernel,
        mesh=_mesh(),
        out_type=jax.ShapeDtypeStruct((E, H), _f32),
        scratch_types=(
            [pltpu.VMEM((2, IB, K2), jnp.int32),
             pltpu.VMEM((2, IB, K2), jnp.int32)]
            + [pltpu.VMEM((K2, H), _f32)] * (2 * NBUF_PAIR)
            + [pltpu.SemaphoreType.DMA] * (2 * NBUF_PAIR)
        ),
    )
    def k(a_hbm, b_hbm, src4_hbm, dst4_hbm, g_hbm, srcb, dstb, *rest):
        ra = rest[:NBUF_PAIR]
        rb = rest[NBUF_PAIR:2 * NBUF_PAIR]
        sg = rest[2 * NBUF_PAIR:3 * NBUF_PAIR]
        sw = rest[3 * NBUF_PAIR:]
        c = lax.axis_index("c")
        s = lax.axis_index("s")
        wid = s * NC + c
        cb0 = wid * TPAIR
        gb0 = wid * (TPAIR // IB)

        def wait_gathers(p, slot, irow):
            pltpu.make_async_copy(
                a_hbm.at[srcb.at[slot, irow]], ra[p], sg[p]).wait()
            pltpu.make_async_copy(
                b_hbm.at[dstb.at[slot, irow]], rb[p], sg[p]).wait()

        def add_and_write(p, kk):
            def add(i, _):
                r = i // (H // 16)
                col = (i % (H // 16)) * 16
                ra[p][r, pl.ds(col, 16)] = (ra[p][r, pl.ds(col, 16)]
                                            + rb[p][r, pl.ds(col, 16)])
                return 0
            lax.fori_loop(0, K2 * H // 16, add, 0, unroll=8)
            base = (cb0 + kk) * K2
            pltpu.async_copy(ra[p], g_hbm.at[pl.ds(base, K2)], sw[p])

        def wait_write(p):
            pltpu.make_async_copy(ra[p], g_hbm.at[pl.ds(0, K2)], sw[p]).wait()

        def group(g, _):
            q = jnp.remainder(g, 2)
            pltpu.sync_copy(src4_hbm.at[gb0 + g], srcb.at[q])
            pltpu.sync_copy(dst4_hbm.at[gb0 + g], dstb.at[q])
            for bq in range(IB):
                kk = g * IB + bq
                p = bq % NBUF_PAIR
                pm1 = (bq - 1) % NBUF_PAIR
                pslot, prow = (q, bq - 1) if bq >= 1 else (1 - q, IB - 1)

                @pl.when(kk >= NBUF_PAIR)
                def _():
                    wait_write(p)
                pltpu.async_copy(a_hbm.at[srcb.at[q, bq]], ra[p], sg[p])
                pltpu.async_copy(b_hbm.at[dstb.at[q, bq]], rb[p], sg[p])

                @pl.when(kk >= 1)
                def _():
                    wait_gathers(pm1, pslot, prow)
                    add_and_write(pm1, kk - 1)
            return 0
        lax.fori_loop(0, TPAIR // IB, group, 0, unroll=False)
        qlast = (TPAIR // IB - 1) % 2
        plast = (TPAIR - 1) % NBUF_PAIR
        wait_gathers(plast, qlast, IB - 1)
        add_and_write(plast, TPAIR - 1)
        for pp in range(NBUF_PAIR):
            wait_write(pp)

    return k(a, b, src4, dst4)


NB = 1000


def _zspec(page):
    return pl.BlockSpec((1, NB, HH), lambda i: (page, i, 0))


def _degspec():
    return pl.BlockSpec((NB, 1), lambda i: (i, 0))


def _silu(x):
    return x * jax.nn.sigmoid(x)


def _tc_encoder_h(x, w1t, b1, g1, bb1, w2t, b2):
    def body(x_r, w1t_r, b1_r, g1_r, bb1_r, w2t_r, b2_r, h_r):
        h = jnp.dot(x_r[...], w1t_r[...], preferred_element_type=_f32) + b1_r[...]
        mu = jnp.mean(h, axis=-1, keepdims=True)
        var = jnp.mean((h - mu) ** 2, axis=-1, keepdims=True)
        h = (h - mu) * lax.rsqrt(var + 1e-5) * g1_r[...] + bb1_r[...]
        h = _silu(h)
        h_r[...] = jnp.dot(h, w2t_r[...], preferred_element_type=_f32) + b2_r[...]

    return pl.pallas_call(
        body,
        grid=(N // NB,),
        in_specs=[
            pl.BlockSpec((NB, D), lambda i: (i, 0)),
            pl.BlockSpec((D, H), lambda i: (0, 0)),
            pl.BlockSpec((1, H), lambda i: (0, 0)),
            pl.BlockSpec((1, H), lambda i: (0, 0)),
            pl.BlockSpec((1, H), lambda i: (0, 0)),
            pl.BlockSpec((H, H), lambda i: (0, 0)),
            pl.BlockSpec((1, H), lambda i: (0, 0)),
        ],
        out_specs=pl.BlockSpec((NB, H), lambda i: (i, 0)),
        out_shape=jax.ShapeDtypeStruct((N, H), _f32),
    )(x, w1t, b1, g1, bb1, w2t, b2)


def _tc_encoder_y(h, dga, dgb, w0t):
    def body(h_r, dga_r, dgb_r, w0t_r, ylo_r, yhi_r):
        dinv = lax.rsqrt(dga_r[...] + dgb_r[...] + 1.0)
        y = dinv * jnp.dot(h_r[...], w0t_r[...], preferred_element_type=_f32)
        ylo_r[...] = y[:, :HH]
        yhi_r[...] = y[:, HH:]

    return pl.pallas_call(
        body,
        grid=(N // NB,),
        in_specs=[
            pl.BlockSpec((NB, H), lambda i: (i, 0)),
            _degspec(),
            _degspec(),
            pl.BlockSpec((H, H), lambda i: (0, 0)),
        ],
        out_specs=[
            pl.BlockSpec((NB, HH), lambda i: (i, 0)),
            pl.BlockSpec((NB, HH), lambda i: (i, 0)),
        ],
        out_shape=[
            jax.ShapeDtypeStruct((N, HH), _f32),
            jax.ShapeDtypeStruct((N, HH), _f32),
        ],
    )(h, dga, dgb, w0t)


def _hcat_zy(zlo_r, zhi_r, ylo_r, yhi_r):
    return jnp.concatenate(
        [zlo_r[...].reshape(NB, HH) + ylo_r[...],
         zhi_r[...].reshape(NB, HH) + yhi_r[...]], axis=-1)


def _tc_mid(dga, dgb, z, ylo, yhi, b_prev, wnt):
    def body(dga_r, dgb_r, zlo_r, zhi_r, ylo_r, yhi_r, b_r, wnt_r,
             olo_r, ohi_r):
        dinv = lax.rsqrt(dga_r[...] + dgb_r[...] + 1.0)
        h = _silu(dinv * _hcat_zy(zlo_r, zhi_r, ylo_r, yhi_r) + b_r[...])
        y = dinv * jnp.dot(h, wnt_r[...], preferred_element_type=_f32)
        olo_r[...] = y[:, :HH]
        ohi_r[...] = y[:, HH:]

    return pl.pallas_call(
        body,
        grid=(N // NB,),
        in_specs=[
            _degspec(),
            _degspec(),
            _zspec(0),
            _zspec(1),
            pl.BlockSpec((NB, HH), lambda i: (i, 0)),
            pl.BlockSpec((NB, HH), lambda i: (i, 0)),
            pl.BlockSpec((1, H), lambda i: (0, 0)),
            pl.BlockSpec((H, H), lambda i: (0, 0)),
        ],
        out_specs=[
            pl.BlockSpec((NB, HH), lambda i: (i, 0)),
            pl.BlockSpec((NB, HH), lambda i: (i, 0)),
        ],
        out_shape=[
            jax.ShapeDtypeStruct((N, HH), _f32),
            jax.ShapeDtypeStruct((N, HH), _f32),
        ],
    )(dga, dgb, z, z, ylo, yhi, b_prev, wnt)


def _tc_final(dga, dgb, z, ylo, yhi, b_prev, w1at, w1bt, pb1):
    def body(dga_r, dgb_r, zlo_r, zhi_r, ylo_r, yhi_r, b_r, wa_r, wb_r, pb1_r,
             a_r, bo_r):
        dinv = lax.rsqrt(dga_r[...] + dgb_r[...] + 1.0)
        h = _silu(dinv * _hcat_zy(zlo_r, zhi_r, ylo_r, yhi_r) + b_r[...])
        a_r[...] = jnp.dot(h, wa_r[...], preferred_element_type=_f32)
        bo_r[...] = jnp.dot(h, wb_r[...], preferred_element_type=_f32) + pb1_r[...]

    return pl.pallas_call(
        body,
        grid=(N // NB,),
        in_specs=[
            _degspec(),
            _degspec(),
            _zspec(0),
            _zspec(1),
            pl.BlockSpec((NB, HH), lambda i: (i, 0)),
            pl.BlockSpec((NB, HH), lambda i: (i, 0)),
            pl.BlockSpec((1, H), lambda i: (0, 0)),
            pl.BlockSpec((H, H), lambda i: (0, 0)),
            pl.BlockSpec((H, H), lambda i: (0, 0)),
            pl.BlockSpec((1, H), lambda i: (0, 0)),
        ],
        out_specs=[
            pl.BlockSpec((NB, H), lambda i: (i, 0)),
            pl.BlockSpec((NB, H), lambda i: (i, 0)),
        ],
        out_shape=[
            jax.ShapeDtypeStruct((N, H), _f32),
            jax.ShapeDtypeStruct((N, H), _f32),
        ],
    )(dga, dgb, z, z, ylo, yhi, b_prev, w1at, w1bt, pb1)


EB = 2000


def _tc_tail(g, png, pnb, w2row, b2):
    def body(g_r, png_r, pnb_r, w2_r, b2_r, o_r):
        q = g_r[...]
        mu = jnp.mean(q, axis=-1, keepdims=True)
        var = jnp.mean((q - mu) ** 2, axis=-1, keepdims=True)
        q = (q - mu) * lax.rsqrt(var + 1e-5) * png_r[...] + pnb_r[...]
        q = _silu(q)
        sres = jnp.sum(q * w2_r[...], axis=-1) + b2_r[0, 0]
        o_r[...] = jax.nn.sigmoid(sres).reshape(EB, 1)

    return pl.pallas_call(
        body,
        grid=(E // EB,),
        in_specs=[
            pl.BlockSpec((EB, H), lambda i: (i, 0)),
            pl.BlockSpec((1, H), lambda i: (0, 0)),
            pl.BlockSpec((1, H), lambda i: (0, 0)),
            pl.BlockSpec((1, H), lambda i: (0, 0)),
            pl.BlockSpec((1, 1), lambda i: (0, 0)),
        ],
        out_specs=pl.BlockSpec((EB, 1), lambda i: (i, 0)),
        out_shape=jax.ShapeDtypeStruct((E, 1), _f32),
    )(g, png, pnb, w2row, b2)


def kernel(node_features, edge_index, enc_W1, enc_b1, ln1_g, ln1_b, enc_W2,
           enc_b2, gcn_W, gcn_b, pred_W1, pred_b1, pln_g, pln_b, pred_W2,
           pred_b2):
    src = edge_index[0]
    dst = edge_index[1]
    src4 = src.reshape(E // (IB * K2), IB, K2)
    dst4 = dst.reshape(E // (IB * K2), IB, K2)
    src2 = src4
    dst2 = dst4
    row = lambda v: v.reshape(1, -1)

    deg_flat = _sc_degree(dst)
    dga = deg_flat[:N].reshape(N, 1)
    dgb = deg_flat[NPAD:NPAD + N].reshape(N, 1)

    h0 = _tc_encoder_h(node_features, enc_W1.T, row(enc_b1), row(ln1_g),
                       row(ln1_b), enc_W2.T, row(enc_b2))
    ylo, yhi = _tc_encoder_y(h0, dga, dgb, gcn_W[0].T)
    for i in range(2):
        z = _sc_propagate(ylo, yhi, src2, dst2)
        ylo, yhi = _tc_mid(dga, dgb, z, ylo, yhi, row(gcn_b[i]),
                           gcn_W[i + 1].T)
    z = _sc_propagate(ylo, yhi, src2, dst2)
    a, b = _tc_final(dga, dgb, z, ylo, yhi, row(gcn_b[2]),
                     pred_W1[:, :H].T, pred_W1[:, H:].T, row(pred_b1))

    g = _sc_pair(a, b, src4, dst4)
    out = _tc_tail(g, row(pln_g), row(pln_b), row(pred_W2[0]),
                   pred_b2.reshape(1, 1))
    return out.reshape(E)


# speedup vs baseline: 1.1528x; 1.1528x over previous
import functools

import jax
import jax.numpy as jnp
from jax import lax
from jax.experimental import pallas as pl
from jax.experimental.pallas import tpu as pltpu
from jax.experimental.pallas import tpu_sc as plsc

N = 10000
E = 320000
D = 128
H = 256

NC = 2
NS = 16
NPAD = 10240
RPW = NPAD // NS
K = 80
EPW_PROP = E // NS
EPW_HALF = E // (NC * NS)

_f32 = jnp.float32


def _mesh():
    return plsc.VectorSubcoreMesh(core_axis_name="c", subcore_axis_name="s")


def _sc_degree(dst):
    @functools.partial(
        pl.kernel,
        mesh=_mesh(),
        out_type=jax.ShapeDtypeStruct((NC * NPAD,), _f32),
        scratch_types=[
            pltpu.VMEM((K,), jnp.int32),
            pltpu.VMEM((K,), _f32),
            pltpu.VMEM((RPW,), _f32),
            pltpu.VMEM_SHARED((NPAD,), _f32),
        ],
    )
    def k(dst_hbm, deg_hbm, dst_v, ones_v, zero_v, acc):
        c = lax.axis_index("c")
        s = lax.axis_index("s")
        wid = s * NC + c

        def fill(i, _):
            ones_v[pl.ds(i * 16, 16)] = jnp.full((16,), 1.0, _f32)
            return 0
        lax.fori_loop(0, K // 16, fill, 0, unroll=False)

        def zf(i, _):
            zero_v[pl.ds(i * 16, 16)] = jnp.zeros((16,), _f32)
            return 0
        lax.fori_loop(0, RPW // 16, zf, 0, unroll=False)
        pltpu.sync_copy(zero_v, acc.at[pl.ds(s * RPW, RPW)])
        plsc.subcore_barrier()

        def chunk(kk, _):
            base = wid * EPW_HALF + kk * K
            pltpu.sync_copy(dst_hbm.at[pl.ds(base, K)], dst_v)
            pltpu.sync_copy(ones_v, acc.at[dst_v], add=True)
            return 0
        lax.fori_loop(0, EPW_HALF // K, chunk, 0, unroll=False)
        plsc.subcore_barrier()
        pltpu.sync_copy(acc.at[pl.ds(s * RPW, RPW)], zero_v)
        pltpu.sync_copy(zero_v, deg_hbm.at[pl.ds(c * NPAD + s * RPW, RPW)])

    return k(dst)


HH = H // 2


IB = 10
KP = 80
NBUF = 2
NBUF_PAIR = 5
TPROP = EPW_PROP // KP


def _sc_propagate(y_lo, y_hi, src2, dst2):
    @functools.partial(
        pl.kernel,
        mesh=_mesh(),
        out_type=jax.ShapeDtypeStruct((NC, NPAD, HH), _f32),
        scratch_types=(
            [pltpu.VMEM((2, IB, KP), jnp.int32),
             pltpu.VMEM((2, IB, KP), jnp.int32)]
            + [pltpu.VMEM((KP, HH), _f32)] * NBUF
            + [pltpu.VMEM_SHARED((NPAD, HH), _f32)]
            + [pltpu.SemaphoreType.DMA] * (2 * NBUF)
        ),
    )
    def k(ylo_hbm, yhi_hbm, src2_hbm, dst2_hbm, z_hbm, srcb, dstb, *rest):
        rows = rest[:NBUF]
        acc = rest[NBUF]
        sg = rest[NBUF + 1:NBUF + 1 + NBUF]
        ss = rest[NBUF + 1 + NBUF:]
        rows0 = rows[0]
        c = lax.axis_index("c")
        s = lax.axis_index("s")

        def zf(i, _):
            rows0[i // 8, pl.ds((i % 8) * 16, 16)] = jnp.zeros((16,), _f32)
            return 0
        lax.fori_loop(0, KP * HH // 16, zf, 0, unroll=8)

        def zcopy(i, _):
            pltpu.sync_copy(rows0, acc.at[pl.ds(s * RPW + i * KP, KP)])
            return 0
        lax.fori_loop(0, RPW // KP, zcopy, 0, unroll=False)
        plsc.subcore_barrier()

        gb0 = s * (TPROP // IB)

        def edge_loop(y_hbm):
            def wait_gather(p, slot, irow):
                pltpu.make_async_copy(
                    y_hbm.at[srcb.at[slot, irow]], rows[p], sg[p]).wait()

            def wait_scatter(p, slot, irow):
                pltpu.make_async_copy(
                    rows[p], acc.at[dstb.at[slot, irow]], ss[p]).wait()

            def group(g, _):
                q = jnp.remainder(g, 2)
                pltpu.sync_copy(src2_hbm.at[gb0 + g], srcb.at[q])
                pltpu.sync_copy(dst2_hbm.at[gb0 + g], dstb.at[q])
                for b in range(IB):
                    kk = g * IB + b
                    p = b % NBUF
                    pm1 = (b - 1) % NBUF
                    pslot, prow = (q, b - 1) if b >= 1 else (1 - q, IB - 1)

                    @pl.when(kk >= NBUF)
                    def _():
                        wait_scatter(p, pslot, prow)
                    pltpu.async_copy(y_hbm.at[srcb.at[q, b]], rows[p], sg[p])

                    @pl.when(kk >= 1)
                    def _():
                        wait_gather(pm1, pslot, prow)
                        pltpu.async_copy(rows[pm1],
                                         acc.at[dstb.at[pslot, prow]],
                                         ss[pm1], add=True)
                return 0
            lax.fori_loop(0, TPROP // IB, group, 0, unroll=False)
            qlast = (TPROP // IB - 1) % 2
            plast = (TPROP - 1) % NBUF
            wait_gather(plast, qlast, IB - 1)
            pltpu.async_copy(rows[plast], acc.at[dstb.at[qlast, IB - 1]],
                             ss[plast], add=True)
            for pp in range(NBUF):
                wait_scatter(pp, qlast, IB - 1)

        @pl.when(c == 0)
        def _():
            edge_loop(ylo_hbm)

        @pl.when(c == 1)
        def _():
            edge_loop(yhi_hbm)

        plsc.subcore_barrier()

        def wout(i, _):
            r0 = s * RPW + i * KP
            pltpu.sync_copy(acc.at[pl.ds(r0, KP)], rows0)
            pltpu.sync_copy(rows0, z_hbm.at[c, pl.ds(r0, KP)])
            return 0
        lax.fori_loop(0, RPW // KP, wout, 0, unroll=False)

    return k(y_lo, y_hi, src2, dst2)


K2 = 40
TPAIR = EPW_HALF // K2


def _sc_pair(a, b, src4, dst4):
    @functools.partial(
        pl.kernel,
        mesh=_mesh(),
        out_type=jax.ShapeDtypeStruct((E, H), _f32),
        scratch_types=(
            [pltpu.VMEM((2, IB, K2), jnp.int32),
             pltpu.VMEM((2, IB, K2), jnp.int32)]
            + [pltpu.VMEM((K2, H), _f32)] * (2 * NBUF_PAIR)
            + [pltpu.SemaphoreType.DMA] * (2 * NBUF_PAIR)
        ),
    )
    def k(a_hbm, b_hbm, src4_hbm, dst4_hbm, g_hbm, srcb, dstb, *rest):
        ra = rest[:NBUF_PAIR]
        rb = rest[NBUF_PAIR:2 * NBUF_PAIR]
        sg = rest[2 * NBUF_PAIR:3 * NBUF_PAIR]
        sw = rest[3 * NBUF_PAIR:]
        c = lax.axis_index("c")
        s = lax.axis_index("s")
        wid = s * NC + c
        cb0 = wid * TPAIR
        gb0 = wid * (TPAIR // IB)

        def wait_gathers(p, slot, irow):
            pltpu.make_async_copy(
                a_hbm.at[srcb.at[slot, irow]], ra[p], sg[p]).wait()
            pltpu.make_async_copy(
                b_hbm.at[dstb.at[slot, irow]], rb[p], sg[p]).wait()

        def add_and_write(p, kk):
            def add(i, _):
                r = i // (H // 16)
                col = (i % (H // 16)) * 16
                ra[p][r, pl.ds(col, 16)] = (ra[p][r, pl.ds(col, 16)]
                                            + rb[p][r, pl.ds(col, 16)])
                return 0
            lax.fori_loop(0, K2 * H // 16, add, 0, unroll=8)
            base = (cb0 + kk) * K2
            pltpu.async_copy(ra[p], g_hbm.at[pl.ds(base, K2)], sw[p])

        def wait_write(p):
            pltpu.make_async_copy(ra[p], g_hbm.at[pl.ds(0, K2)], sw[p]).wait()

        def group(g, _):
            q = jnp.remainder(g, 2)
            pltpu.sync_copy(src4_hbm.at[gb0 + g], srcb.at[q])
            pltpu.sync_copy(dst4_hbm.at[gb0 + g], dstb.at[q])
            for bq in range(IB):
                kk = g * IB + bq
                p = bq % NBUF_PAIR
                pm1 = (bq - 1) % NBUF_PAIR
                pslot, prow = (q, bq - 1) if bq >= 1 else (1 - q, IB - 1)

                @pl.when(kk >= NBUF_PAIR)
                def _():
                    wait_write(p)
                pltpu.async_copy(a_hbm.at[srcb.at[q, bq]], ra[p], sg[p])
                pltpu.async_copy(b_hbm.at[dstb.at[q, bq]], rb[p], sg[p])

                @pl.when(kk >= 1)
                def _():
                    wait_gathers(pm1, pslot, prow)
                    add_and_write(pm1, kk - 1)
            return 0
        lax.fori_loop(0, TPAIR // IB, group, 0, unroll=False)
        qlast = (TPAIR // IB - 1) % 2
        plast = (TPAIR - 1) % NBUF_PAIR
        wait_gathers(plast, qlast, IB - 1)
        add_and_write(plast, TPAIR - 1)
        for pp in range(NBUF_PAIR):
            wait_write(pp)

    return k(a, b, src4, dst4)


NB = 1000


def _zspec(page):
    return pl.BlockSpec((1, NB, HH), lambda i: (page, i, 0))


def _degspec():
    return pl.BlockSpec((NB, 1), lambda i: (i, 0))


def _silu(x):
    return x * jax.nn.sigmoid(x)


def _tc_encoder_h(x, w1t, b1, g1, bb1, w2t, b2):
    def body(x_r, w1t_r, b1_r, g1_r, bb1_r, w2t_r, b2_r, h_r):
        h = jnp.dot(x_r[...], w1t_r[...], preferred_element_type=_f32) + b1_r[...]
        mu = jnp.mean(h, axis=-1, keepdims=True)
        var = jnp.mean((h - mu) ** 2, axis=-1, keepdims=True)
        h = (h - mu) * lax.rsqrt(var + 1e-5) * g1_r[...] + bb1_r[...]
        h = _silu(h)
        h_r[...] = jnp.dot(h, w2t_r[...], preferred_element_type=_f32) + b2_r[...]

    return pl.pallas_call(
        body,
        grid=(N // NB,),
        in_specs=[
            pl.BlockSpec((NB, D), lambda i: (i, 0)),
            pl.BlockSpec((D, H), lambda i: (0, 0)),
            pl.BlockSpec((1, H), lambda i: (0, 0)),
            pl.BlockSpec((1, H), lambda i: (0, 0)),
            pl.BlockSpec((1, H), lambda i: (0, 0)),
            pl.BlockSpec((H, H), lambda i: (0, 0)),
            pl.BlockSpec((1, H), lambda i: (0, 0)),
        ],
        out_specs=pl.BlockSpec((NB, H), lambda i: (i, 0)),
        out_shape=jax.ShapeDtypeStruct((N, H), _f32),
    )(x, w1t, b1, g1, bb1, w2t, b2)


def _tc_encoder_y(h, dga, dgb, w0t):
    def body(h_r, dga_r, dgb_r, w0t_r, ylo_r, yhi_r):
        dinv = lax.rsqrt(dga_r[...] + dgb_r[...] + 1.0)
        y = dinv * jnp.dot(h_r[...], w0t_r[...], preferred_element_type=_f32)
        ylo_r[...] = y[:, :HH]
        yhi_r[...] = y[:, HH:]

    return pl.pallas_call(
        body,
        grid=(N // NB,),
        in_specs=[
            pl.BlockSpec((NB, H), lambda i: (i, 0)),
            _degspec(),
            _degspec(),
            pl.BlockSpec((H, H), lambda i: (0, 0)),
        ],
        out_specs=[
            pl.BlockSpec((NB, HH), lambda i: (i, 0)),
            pl.BlockSpec((NB, HH), lambda i: (i, 0)),
        ],
        out_shape=[
            jax.ShapeDtypeStruct((N, HH), _f32),
            jax.ShapeDtypeStruct((N, HH), _f32),
        ],
    )(h, dga, dgb, w0t)


def _hcat_zy(zlo_r, zhi_r, ylo_r, yhi_r):
    return jnp.concatenate(
        [zlo_r[...].reshape(NB, HH) + ylo_r[...],
         zhi_r[...].reshape(NB, HH) + yhi_r[...]], axis=-1)


def _tc_mid(dga, dgb, z, ylo, yhi, b_prev, wnt):
    def body(dga_r, dgb_r, zlo_r, zhi_r, ylo_r, yhi_r, b_r, wnt_r,
             olo_r, ohi_r):
        dinv = lax.rsqrt(dga_r[...] + dgb_r[...] + 1.0)
        h = _silu(dinv * _hcat_zy(zlo_r, zhi_r, ylo_r, yhi_r) + b_r[...])
        y = dinv * jnp.dot(h, wnt_r[...], preferred_element_type=_f32)
        olo_r[...] = y[:, :HH]
        ohi_r[...] = y[:, HH:]

    return pl.pallas_call(
        body,
        grid=(N // NB,),
        in_specs=[
            _degspec(),
            _degspec(),
            _zspec(0),
            _zspec(1),
            pl.BlockSpec((NB, HH), lambda i: (i, 0)),
            pl.BlockSpec((NB, HH), lambda i: (i, 0)),
            pl.BlockSpec((1, H), lambda i: (0, 0)),
            pl.BlockSpec((H, H), lambda i: (0, 0)),
        ],
        out_specs=[
            pl.BlockSpec((NB, HH), lambda i: (i, 0)),
            pl.BlockSpec((NB, HH), lambda i: (i, 0)),
        ],
        out_shape=[
            jax.ShapeDtypeStruct((N, HH), _f32),
            jax.ShapeDtypeStruct((N, HH), _f32),
        ],
    )(dga, dgb, z, z, ylo, yhi, b_prev, wnt)


def _tc_final(dga, dgb, z, ylo, yhi, b_prev, w1at, w1bt, pb1):
    def body(dga_r, dgb_r, zlo_r, zhi_r, ylo_r, yhi_r, b_r, wa_r, wb_r, pb1_r,
             a_r, bo_r):
        dinv = lax.rsqrt(dga_r[...] + dgb_r[...] + 1.0)
        h = _silu(dinv * _hcat_zy(zlo_r, zhi_r, ylo_r, yhi_r) + b_r[...])
        a_r[...] = jnp.dot(h, wa_r[...], preferred_element_type=_f32)
        bo_r[...] = jnp.dot(h, wb_r[...], preferred_element_type=_f32) + pb1_r[...]

    return pl.pallas_call(
        body,
        grid=(N // NB,),
        in_specs=[
            _degspec(),
            _degspec(),
            _zspec(0),
            _zspec(1),
            pl.BlockSpec((NB, HH), lambda i: (i, 0)),
            pl.BlockSpec((NB, HH), lambda i: (i, 0)),
            pl.BlockSpec((1, H), lambda i: (0, 0)),
            pl.BlockSpec((H, H), lambda i: (0, 0)),
            pl.BlockSpec((H, H), lambda i: (0, 0)),
            pl.BlockSpec((1, H), lambda i: (0, 0)),
        ],
        out_specs=[
            pl.BlockSpec((NB, H), lambda i: (i, 0)),
            pl.BlockSpec((NB, H), lambda i: (i, 0)),
        ],
        out_shape=[
            jax.ShapeDtypeStruct((N, H), _f32),
            jax.ShapeDtypeStruct((N, H), _f32),
        ],
    )(dga, dgb, z, z, ylo, yhi, b_prev, w1at, w1bt, pb1)


EB = 2000


def _tc_tail(g, png, pnb, w2row, b2):
    def body(g_r, png_r, pnb_r, w2_r, b2_r, o_r):
        q = g_r[...]
        mu = jnp.mean(q, axis=-1, keepdims=True)
        var = jnp.mean((q - mu) ** 2, axis=-1, keepdims=True)
        q = (q - mu) * lax.rsqrt(var + 1e-5) * png_r[...] + pnb_r[...]
        q = _silu(q)
        sres = jnp.sum(q * w2_r[...], axis=-1) + b2_r[0, 0]
        o_r[...] = jax.nn.sigmoid(sres).reshape(EB, 1)

    return pl.pallas_call(
        body,
        grid=(E // EB,),
        in_specs=[
            pl.BlockSpec((EB, H), lambda i: (i, 0)),
            pl.BlockSpec((1, H), lambda i: (0, 0)),
            pl.BlockSpec((1, H), lambda i: (0, 0)),
            pl.BlockSpec((1, H), lambda i: (0, 0)),
            pl.BlockSpec((1, 1), lambda i: (0, 0)),
        ],
        out_specs=pl.BlockSpec((EB, 1), lambda i: (i, 0)),
        out_shape=jax.ShapeDtypeStruct((E, 1), _f32),
    )(g, png, pnb, w2row, b2)


def kernel(node_features, edge_index, enc_W1, enc_b1, ln1_g, ln1_b, enc_W2,
           enc_b2, gcn_W, gcn_b, pred_W1, pred_b1, pln_g, pln_b, pred_W2,
           pred_b2):
    src = edge_index[0]
    dst = edge_index[1]
    src4 = src.reshape(E // (IB * K2), IB, K2)
    dst4 = dst.reshape(E // (IB * K2), IB, K2)
    src2 = src.reshape(E // (IB * KP), IB, KP)
    dst2 = dst.reshape(E // (IB * KP), IB, KP)
    row = lambda v: v.reshape(1, -1)

    deg_flat = _sc_degree(dst)
    dga = deg_flat[:N].reshape(N, 1)
    dgb = deg_flat[NPAD:NPAD + N].reshape(N, 1)

    h0 = _tc_encoder_h(node_features, enc_W1.T, row(enc_b1), row(ln1_g),
                       row(ln1_b), enc_W2.T, row(enc_b2))
    ylo, yhi = _tc_encoder_y(h0, dga, dgb, gcn_W[0].T)
    for i in range(2):
        z = _sc_propagate(ylo, yhi, src2, dst2)
        ylo, yhi = _tc_mid(dga, dgb, z, ylo, yhi, row(gcn_b[i]),
                           gcn_W[i + 1].T)
    z = _sc_propagate(ylo, yhi, src2, dst2)
    a, b = _tc_final(dga, dgb, z, ylo, yhi, row(gcn_b[2]),
                     pred_W1[:, :H].T, pred_W1[:, H:].T, row(pred_b1))

    g = _sc_pair(a, b, src4, dst4)
    out = _tc_tail(g, row(pln_g), row(pln_b), row(pred_W2[0]),
                   pred_b2.reshape(1, 1))
    return out.reshape(E)
